# baseline (speedup 1.0000x reference)
import functools

import jax
import jax.numpy as jnp
from jax import lax
from jax.experimental import pallas as pl
from jax.experimental.pallas import tpu as pltpu
from jax.experimental.pallas import tpu_sc as plsc

N_CODES = 1024
DIM = 64
N_TOK = 64 * 576
BR = 9216
N_BLK = N_TOK // BR

SC_NC = 2
SC_NW = 32
B_PER_W = N_TOK // SC_NW
IDX_CH = 128
N_CH = B_PER_W // IDX_CH


def _dist_argmin_body(x_ref, c_ref, cn_ref, idx_ref, loss_ref):
    i = pl.program_id(0)
    x = x_ref[...]
    c = c_ref[...]
    xn = jnp.sum(x * x, axis=1, keepdims=True)
    m2 = lax.dot_general(x + x, c, (((1,), (1,)), ((), ())),
                         preferred_element_type=jnp.float32)
    d = (xn + cn_ref[...]) - m2
    min_d = jnp.min(d, axis=1, keepdims=True)
    code_iota = lax.broadcasted_iota(
        jnp.int32, (1, N_CODES), 1).astype(jnp.float32)
    idx = jnp.min(jnp.where(d == min_d, code_iota, jnp.float32(N_CODES)),
                  axis=1, keepdims=True)
    idx_ref[...] = idx.astype(jnp.int32)

    @pl.when(i == 0)
    def _():
        loss_ref[0, 0] = 0.0

    loss_ref[0, 0] += jnp.sum(min_d)


def _dist_argmin(flat, codebook, cn):
    return pl.pallas_call(
        _dist_argmin_body,
        grid=(N_BLK,),
        in_specs=[
            pl.BlockSpec((BR, DIM), lambda i: (i, 0)),
            pl.BlockSpec((N_CODES, DIM), lambda i: (0, 0)),
            pl.BlockSpec((1, N_CODES), lambda i: (0, 0)),
        ],
        out_specs=[
            pl.BlockSpec((BR, 1), lambda i: (i, 0)),
            pl.BlockSpec(memory_space=pltpu.SMEM),
        ],
        out_shape=[
            jax.ShapeDtypeStruct((N_TOK, 1), jnp.int32),
            jax.ShapeDtypeStruct((1, 1), jnp.float32),
        ],
        compiler_params=pltpu.CompilerParams(
            dimension_semantics=("arbitrary",),
            vmem_limit_bytes=128 * 1024 * 1024),
    )(flat, codebook, cn)


def _sc_gather_body(cb_hbm, idx_hbm, out_hbm, idx_v, rows_v, sem):
    wid = lax.axis_index("s") * SC_NC + lax.axis_index("c")
    base = wid * B_PER_W
    pltpu.sync_copy(idx_hbm.at[pl.ds(base, B_PER_W)], idx_v)
    copies = [
        pltpu.async_copy(cb_hbm.at[idx_v.at[pl.ds(k * IDX_CH, IDX_CH)]],
                         rows_v.at[pl.ds(k * IDX_CH, IDX_CH)], sem)
        for k in range(N_CH)
    ]
    for cp in copies:
        cp.wait()
    pltpu.sync_copy(rows_v, out_hbm.at[pl.ds(base, B_PER_W)])


_sc_gather = functools.partial(
    pl.kernel,
    out_type=jax.ShapeDtypeStruct((N_TOK, DIM), jnp.float32),
    mesh=plsc.VectorSubcoreMesh(core_axis_name="c", subcore_axis_name="s"),
    scratch_types=[
        pltpu.VMEM((B_PER_W,), jnp.int32),
        pltpu.VMEM((B_PER_W, DIM), jnp.float32),
        pltpu.SemaphoreType.DMA,
    ],
    compiler_params=pltpu.CompilerParams(use_tc_tiling_on_sc=False),
)(_sc_gather_body)


def kernel(inputs, codebook, beta):
    flat = inputs.reshape(-1, DIM)
    cn = jnp.sum(codebook ** 2, axis=1)[None, :]
    idx2d, loss_sum = _dist_argmin(flat, codebook, cn)
    idx = idx2d.reshape(-1)
    quantized = _sc_gather(codebook, idx)
    quantized_st = quantized.reshape(inputs.shape)
    mean_sq = loss_sum[0, 0] / jnp.float32(N_TOK * DIM)
    commitment_loss = mean_sq + beta * mean_sq
    return quantized_st, commitment_loss, idx


# speedup vs baseline: 1.0062x; 1.0062x over previous
import functools

import jax
import jax.numpy as jnp
from jax import lax
from jax.experimental import pallas as pl
from jax.experimental.pallas import tpu as pltpu
from jax.experimental.pallas import tpu_sc as plsc

N_CODES = 1024
DIM = 64
N_TOK = 64 * 576
BR = 6144
N_BLK = N_TOK // BR

SC_NC = 2
SC_NW = 32
B_PER_W = N_TOK // SC_NW
IDX_CH = 128
N_CH = B_PER_W // IDX_CH


def _dist_argmin_body(x_ref, c_ref, cn_ref, idx_ref, loss_ref):
    i = pl.program_id(0)
    x = x_ref[...]
    c = c_ref[...]
    xn = jnp.sum(x * x, axis=1, keepdims=True)
    m2 = lax.dot_general(x + x, c, (((1,), (1,)), ((), ())),
                         preferred_element_type=jnp.float32)
    d = (xn + cn_ref[...]) - m2
    min_d = jnp.min(d, axis=1, keepdims=True)
    code_iota = lax.broadcasted_iota(
        jnp.int32, (1, N_CODES), 1).astype(jnp.float32)
    idx = jnp.min(jnp.where(d == min_d, code_iota, jnp.float32(N_CODES)),
                  axis=1, keepdims=True)
    idx_ref[...] = idx.astype(jnp.int32)

    @pl.when(i == 0)
    def _():
        loss_ref[0, 0] = 0.0

    loss_ref[0, 0] += jnp.sum(min_d)


def _dist_argmin(flat, codebook, cn):
    return pl.pallas_call(
        _dist_argmin_body,
        grid=(N_BLK,),
        in_specs=[
            pl.BlockSpec((BR, DIM), lambda i: (i, 0)),
            pl.BlockSpec((N_CODES, DIM), lambda i: (0, 0)),
            pl.BlockSpec((1, N_CODES), lambda i: (0, 0)),
        ],
        out_specs=[
            pl.BlockSpec((BR, 1), lambda i: (i, 0)),
            pl.BlockSpec(memory_space=pltpu.SMEM),
        ],
        out_shape=[
            jax.ShapeDtypeStruct((N_TOK, 1), jnp.int32),
            jax.ShapeDtypeStruct((1, 1), jnp.float32),
        ],
        compiler_params=pltpu.CompilerParams(
            dimension_semantics=("arbitrary",)),
    )(flat, codebook, cn)


def _sc_gather_body(cb_hbm, idx_hbm, out_hbm, idx_v, rows_v, sem):
    wid = lax.axis_index("s") * SC_NC + lax.axis_index("c")
    base = wid * B_PER_W
    pltpu.sync_copy(idx_hbm.at[pl.ds(base, B_PER_W)], idx_v)
    copies = [
        pltpu.async_copy(cb_hbm.at[idx_v.at[pl.ds(k * IDX_CH, IDX_CH)]],
                         rows_v.at[pl.ds(k * IDX_CH, IDX_CH)], sem)
        for k in range(N_CH)
    ]
    for cp in copies:
        cp.wait()
    pltpu.sync_copy(rows_v, out_hbm.at[pl.ds(base, B_PER_W)])


_sc_gather = functools.partial(
    pl.kernel,
    out_type=jax.ShapeDtypeStruct((N_TOK, DIM), jnp.float32),
    mesh=plsc.VectorSubcoreMesh(core_axis_name="c", subcore_axis_name="s"),
    scratch_types=[
        pltpu.VMEM((B_PER_W,), jnp.int32),
        pltpu.VMEM((B_PER_W, DIM), jnp.float32),
        pltpu.SemaphoreType.DMA,
    ],
    compiler_params=pltpu.CompilerParams(use_tc_tiling_on_sc=False),
)(_sc_gather_body)


def kernel(inputs, codebook, beta):
    flat = inputs.reshape(-1, DIM)
    cn = jnp.sum(codebook ** 2, axis=1)[None, :]
    idx2d, loss_sum = _dist_argmin(flat, codebook, cn)
    idx = idx2d.reshape(-1)
    quantized = _sc_gather(codebook, idx)
    quantized_st = quantized.reshape(inputs.shape)
    mean_sq = loss_sum[0, 0] / jnp.float32(N_TOK * DIM)
    commitment_loss = mean_sq + beta * mean_sq
    return quantized_st, commitment_loss, idx
